# baseline (speedup 1.0000x reference)
import jax
import jax.numpy as jnp
from jax import lax
from jax.experimental import pallas as pl
from jax.experimental.pallas import tpu as pltpu

_SIZE = 65536
_DIM = 128
_BATCH = 4096
_R = 2048
_NB = _SIZE // _R
_WINB = _BATCH // _R


def _body(p_ref, q_hbm, f_hbm, o_hbm, np_ref, sems):
    p_blk = p_ref[0] // _R

    for b in range(_NB):
        j = lax.rem(b - p_blk + _NB, _NB)
        dst = o_hbm.at[pl.ds(b * _R, _R)]

        @pl.when(j < _WINB)
        def _(j=j, dst=dst, b=b):
            pltpu.make_async_copy(
                f_hbm.at[pl.ds(j * _R, _R)], dst, sems.at[b]).start()

        @pl.when(j >= _WINB)
        def _(dst=dst, b=b):
            pltpu.make_async_copy(
                q_hbm.at[pl.ds(b * _R, _R)], dst, sems.at[b]).start()

    np_ref[0] = lax.rem(p_ref[0] + _BATCH, _SIZE)

    for b in range(_NB):
        pltpu.make_async_copy(
            q_hbm.at[pl.ds(b * _R, _R)],
            o_hbm.at[pl.ds(b * _R, _R)],
            sems.at[b]).wait()


def _run(p_arr, queue, feats):
    grid_spec = pltpu.PrefetchScalarGridSpec(
        num_scalar_prefetch=1,
        grid=(1,),
        in_specs=[
            pl.BlockSpec(memory_space=pl.ANY),
            pl.BlockSpec(memory_space=pl.ANY),
        ],
        out_specs=[
            pl.BlockSpec(memory_space=pl.ANY),
            pl.BlockSpec(memory_space=pltpu.SMEM),
        ],
        scratch_shapes=[pltpu.SemaphoreType.DMA((_NB,))],
    )
    return pl.pallas_call(
        _body,
        grid_spec=grid_spec,
        out_shape=[
            jax.ShapeDtypeStruct((_SIZE, _DIM), jnp.float32),
            jax.ShapeDtypeStruct((1,), jnp.int32),
        ],
        compiler_params=pltpu.CompilerParams(
            dimension_semantics=("arbitrary",),
        ),
    )(p_arr, queue, feats)


def kernel(queue, feats, ptr):
    p_arr = jnp.reshape(ptr, (1,)).astype(jnp.int32)
    new_queue, new_ptr = _run(p_arr, queue, feats)
    return new_queue, new_ptr


# speedup vs baseline: 38.8427x; 38.8427x over previous
import jax
import jax.numpy as jnp
from jax import lax
from jax.experimental import pallas as pl
from jax.experimental.pallas import tpu as pltpu

_SIZE = 65536
_DIM = 128
_BATCH = 4096
_R = 2048
_NB = _SIZE // _R
_WINB = _BATCH // _R


def _body(p_ref, q_ref, f_ref, o_ref, np_ref):
    o_ref[...] = f_ref[...]

    @pl.when(pl.program_id(0) == 0)
    def _():
        np_ref[0] = lax.rem(p_ref[0] + _BATCH, _SIZE)


def _o_map(i, p_ref):
    p_blk = p_ref[0] // _R
    return lax.rem(p_blk + i, _NB), 0


def _run(p_arr, queue, feats):
    grid_spec = pltpu.PrefetchScalarGridSpec(
        num_scalar_prefetch=1,
        grid=(_WINB,),
        in_specs=[
            pl.BlockSpec(memory_space=pl.ANY),
            pl.BlockSpec((_R, _DIM), lambda i, p: (i, 0)),
        ],
        out_specs=[
            pl.BlockSpec((_R, _DIM), _o_map),
            pl.BlockSpec(memory_space=pltpu.SMEM),
        ],
    )
    return pl.pallas_call(
        _body,
        grid_spec=grid_spec,
        out_shape=[
            jax.ShapeDtypeStruct((_SIZE, _DIM), jnp.float32),
            jax.ShapeDtypeStruct((1,), jnp.int32),
        ],
        input_output_aliases={1: 0},
        compiler_params=pltpu.CompilerParams(
            dimension_semantics=("arbitrary",),
        ),
    )(p_arr, queue, feats)


def kernel(queue, feats, ptr):
    p_arr = jnp.reshape(ptr, (1,)).astype(jnp.int32)
    new_queue, new_ptr = _run(p_arr, queue, feats)
    return new_queue, new_ptr
